# Initial kernel scaffold; baseline (speedup 1.0000x reference)
import jax
import jax.numpy as jnp
from jax.experimental import pallas as pl


def kernel(p):
    raise NotImplementedError("write your pallas kernel here")


# speedup vs baseline: 1.0379x; 1.0379x over previous
import jax
import jax.numpy as jnp
from jax.experimental import pallas as pl

_NA = 3
_NC = 80
_NO = _NC + 5


def _transpose_body(in_ref, out_ref):
    out_ref[...] = jnp.transpose(in_ref[...], (0, 2, 1))


def kernel(p):
    bs, c, ny, nx = p.shape
    s = ny * nx
    x = p.reshape(bs * _NA, _NO, s)

    out = pl.pallas_call(
        _transpose_body,
        grid=(bs * _NA,),
        in_specs=[pl.BlockSpec((1, _NO, s), lambda i: (i, 0, 0))],
        out_specs=pl.BlockSpec((1, s, _NO), lambda i: (i, 0, 0)),
        out_shape=jax.ShapeDtypeStruct((bs * _NA, s, _NO), jnp.float32),
    )(x)
    return out.reshape(bs, _NA, ny, nx, _NO)
